# Initial kernel scaffold; baseline (speedup 1.0000x reference)
import jax
import jax.numpy as jnp
from jax.experimental import pallas as pl


def kernel(x, pos_emb):
    raise NotImplementedError("write your pallas kernel here")


# speedup vs baseline: 6.1620x; 6.1620x over previous
import jax
import jax.numpy as jnp
from jax.experimental import pallas as pl


def _add_body(x_ref, pe_ref, o_ref):
    o_ref[...] = x_ref[...] + pe_ref[...]


def kernel(x, pos_emb):
    B, L, D = x.shape
    LD = L * D
    x2 = x.reshape(B, LD)
    pe2 = pos_emb[:L].reshape(1, LD)
    BB = 128
    out = pl.pallas_call(
        _add_body,
        grid=(B // BB,),
        in_specs=[
            pl.BlockSpec((BB, LD), lambda i: (i, 0)),
            pl.BlockSpec((1, LD), lambda i: (0, 0)),
        ],
        out_specs=pl.BlockSpec((BB, LD), lambda i: (i, 0)),
        out_shape=jax.ShapeDtypeStruct((B, LD), x.dtype),
    )(x2, pe2)
    return out.reshape(B, L, D)
